# baseline (speedup 1.0000x reference)
import functools

import jax
import jax.numpy as jnp
from jax import lax
from jax.experimental import pallas as pl
from jax.experimental.pallas import tpu as pltpu
from jax.experimental.pallas import tpu_sc as plsc

N = 10000
E = 320000
H = 128
G = 64
L = 3

NC = 2
NS = 16
NW = NC * NS
CHUNK = 128
CH = 80
E_PAD = NW * CH * CHUNK
NP = 10240
HN = NP // 2
ACC_R = HN + 128
LCAP = (CH + 1) * CHUNK + 16
CAPC = 2608
SENT = 1 << 20
DUM = HN
BLK = 1024
GRID = NP // BLK

_sc_mesh = plsc.VectorSubcoreMesh(core_axis_name="c", subcore_axis_name="s")


NCH = NW * CH
CPT = NCH // NS
NGRP = CPT // 16


@functools.partial(
    pl.kernel,
    out_type=jax.ShapeDtypeStruct((NC, ACC_R, H), jnp.float32),
    mesh=_sc_mesh,
    scratch_types=[
        pltpu.VMEM((16, CHUNK), jnp.int32),
        pltpu.VMEM((16, CHUNK), jnp.int32),
        pltpu.VMEM((2, CHUNK, H), jnp.float32),
        pltpu.VMEM_SHARED((HN, H), jnp.float32),
        pltpu.VMEM_SHARED((ACC_R, H), jnp.float32),
        pltpu.SemaphoreType.DMA,
        pltpu.SemaphoreType.DMA,
    ],
)
def _edge_agg(m_hbm, src_hbm, dst_hbm, out_hbm,
              sg_v, dg_v, rows_v, m_sh, acc_sh, sem0, sem1):
    k = lax.axis_index("c")
    s = lax.axis_index("s")
    sems = (sem0, sem1)

    zeros16 = jnp.zeros((16,), jnp.float32)

    def _zero_row(i, carry):
        for j in range(H // 16):
            rows_v[0, i, pl.ds(j * 16, 16)] = zeros16
        return carry

    lax.fori_loop(0, CHUNK, _zero_row, 0)
    zbase = s * (ACC_R // NS)
    for off, nrows in ((0, 128), (128, 128), (256, ACC_R // NS - 256)):
        pltpu.sync_copy(rows_v.at[0, pl.ds(0, nrows)],
                        acc_sh.at[pl.ds(zbase + off, nrows)])

    for p in range(2):
        a_t = k * (1 - p) + (1 - k) * p
        mrows = HN // NS
        pltpu.sync_copy(m_hbm.at[pl.ds(a_t * HN + s * mrows, mrows)],
                        m_sh.at[pl.ds(s * mrows, mrows)])
        plsc.subcore_barrier()

        def _group(g, carry, p=p):
            gbase = s * CPT + g * 16
            pltpu.sync_copy(src_hbm.at[pl.ds(gbase, 16)], sg_v)
            pltpu.sync_copy(dst_hbm.at[pl.ds(gbase, 16)], dg_v)

            for kk in range(NC):
                @pl.when(k == kk)
                def _(kk=kk):
                    a = kk ^ p
                    b = kk

                    def _fix(ch, cy):
                        for j in range(H // 16):
                            sv = sg_v[ch, pl.ds(j * 16, 16)]
                            dv = dg_v[ch, pl.ds(j * 16, 16)]
                            s_lt = lax.shift_right_logical(sv - HN, 31)
                            d_lt = lax.shift_right_logical(dv - HN, 31)
                            valid = lax.shift_right_logical(dv - NP, 31)
                            ma = (1 - s_lt) if a else s_lt
                            mb = (1 - d_lt) if b else d_lt
                            mi = ma * mb * valid
                            ni = 1 - mi
                            sg_v[ch, pl.ds(j * 16, 16)] = (
                                (sv - a * HN) * mi + (sv & 4095) * ni)
                            dg_v[ch, pl.ds(j * 16, 16)] = (
                                (dv - b * HN) * mi + (DUM + (dv & 15)) * ni)
                        return cy

                    lax.fori_loop(0, 16, _fix, 0)

            pltpu.async_copy(m_sh.at[sg_v.at[0]], rows_v.at[0], sem0)
            pltpu.async_copy(m_sh.at[sg_v.at[1]], rows_v.at[1], sem1)

            def _pair(jj, carry2):
                for b in range(2):
                    j = 2 * jj + b
                    pltpu.make_async_copy(m_sh.at[sg_v.at[j]],
                                          rows_v.at[b], sems[b]).wait()
                    pltpu.sync_copy(rows_v.at[b], acc_sh.at[dg_v.at[j]],
                                    add=True)

                    @pl.when(j + 2 < 16)
                    def _():
                        pltpu.async_copy(m_sh.at[sg_v.at[j + 2]],
                                         rows_v.at[b], sems[b])

                return carry2

            lax.fori_loop(0, 8, _pair, 0)
            return carry

        lax.fori_loop(0, NGRP, _group, 0)
        plsc.subcore_barrier()

    nrows = ACC_R // NS
    pltpu.sync_copy(acc_sh.at[pl.ds(s * nrows, nrows)],
                    out_hbm.at[k, pl.ds(s * nrows, nrows)])


def _mm_body(x_ref, w_ref, o_ref):
    o_ref[...] = jnp.dot(x_ref[...], w_ref[...],
                         preferred_element_type=jnp.float32)


_mm = pl.pallas_call(
    _mm_body,
    grid=(GRID,),
    in_specs=[
        pl.BlockSpec((BLK, H), lambda i: (i, 0)),
        pl.BlockSpec((H, H), lambda i: (0, 0)),
    ],
    out_specs=pl.BlockSpec((BLK, H), lambda i: (i, 0)),
    out_shape=jax.ShapeDtypeStruct((NP, H), jnp.float32),
)


def _gru(agg, h, wih, whh, bih, bhh):
    gi = jnp.dot(agg, wih, preferred_element_type=jnp.float32) + bih
    gh = jnp.dot(h, whh, preferred_element_type=jnp.float32) + bhh
    r = jax.nn.sigmoid(gi[:, :H] + gh[:, :H])
    z = jax.nn.sigmoid(gi[:, H:2 * H] + gh[:, H:2 * H])
    n = jnp.tanh(gi[:, 2 * H:] + r * gh[:, 2 * H:])
    return (1.0 - z) * n + z * h


_parts_spec = pl.BlockSpec((1, BLK, H), lambda i: (i // (GRID // 2),
                                                   i % (GRID // 2), 0))


def _gru_mm_body(p_ref, h_ref, wih_ref, whh_ref, bih_ref, bhh_ref, wn_ref,
                 hn_ref, mn_ref):
    hn = _gru(p_ref[0], h_ref[...], wih_ref[...], whh_ref[...],
              bih_ref[...], bhh_ref[...])
    hn_ref[...] = hn
    mn_ref[...] = jnp.dot(hn, wn_ref[...], preferred_element_type=jnp.float32)


_gru_mm = pl.pallas_call(
    _gru_mm_body,
    grid=(GRID,),
    in_specs=[
        _parts_spec,
        pl.BlockSpec((BLK, H), lambda i: (i, 0)),
        pl.BlockSpec((H, 3 * H), lambda i: (0, 0)),
        pl.BlockSpec((H, 3 * H), lambda i: (0, 0)),
        pl.BlockSpec((1, 3 * H), lambda i: (0, 0)),
        pl.BlockSpec((1, 3 * H), lambda i: (0, 0)),
        pl.BlockSpec((H, H), lambda i: (0, 0)),
    ],
    out_specs=[
        pl.BlockSpec((BLK, H), lambda i: (i, 0)),
        pl.BlockSpec((BLK, H), lambda i: (i, 0)),
    ],
    out_shape=[
        jax.ShapeDtypeStruct((NP, H), jnp.float32),
        jax.ShapeDtypeStruct((NP, H), jnp.float32),
    ],
)


def _gru_ro_body(p_ref, h_ref, wih_ref, whh_ref, bih_ref, bhh_ref, b_ref,
                 out_ref):
    hn = _gru(p_ref[0], h_ref[...], wih_ref[...], whh_ref[...],
              bih_ref[...], bhh_ref[...])
    bid = b_ref[0, 0, :]
    oh = (bid[:, None] == lax.broadcasted_iota(jnp.int32, (BLK, G), 1)
          ).astype(jnp.float32)
    contrib = lax.dot_general(oh, hn, (((0,), (0,)), ((), ())),
                              preferred_element_type=jnp.float32)

    @pl.when(pl.program_id(0) == 0)
    def _():
        out_ref[...] = contrib

    @pl.when(pl.program_id(0) > 0)
    def _():
        out_ref[...] += contrib


_gru_ro = pl.pallas_call(
    _gru_ro_body,
    grid=(GRID,),
    in_specs=[
        _parts_spec,
        pl.BlockSpec((BLK, H), lambda i: (i, 0)),
        pl.BlockSpec((H, 3 * H), lambda i: (0, 0)),
        pl.BlockSpec((H, 3 * H), lambda i: (0, 0)),
        pl.BlockSpec((1, 3 * H), lambda i: (0, 0)),
        pl.BlockSpec((1, 3 * H), lambda i: (0, 0)),
        pl.BlockSpec((1, 1, BLK), lambda i: (i, 0, 0)),
    ],
    out_specs=pl.BlockSpec((G, H), lambda i: (0, 0)),
    out_shape=jax.ShapeDtypeStruct((G, H), jnp.float32),
)


def kernel(x, edge_index, batch, weight, W_ih, W_hh, b_ih, b_hh):
    src = edge_index[0].astype(jnp.int32)
    dst = edge_index[1].astype(jnp.int32)
    pad = E_PAD - E
    src2d = jnp.concatenate([src, jnp.zeros((pad,), jnp.int32)]
                            ).reshape(NW * CH, CHUNK)
    dst2d = jnp.concatenate([dst, jnp.full((pad,), SENT, jnp.int32)]
                            ).reshape(NW * CH, CHUNK)
    batch3d = jnp.concatenate([batch.astype(jnp.int32),
                               jnp.full((NP - N,), G, jnp.int32)]
                              ).reshape(GRID, 1, BLK)

    wih = W_ih.T
    whh = W_hh.T
    bih = b_ih.reshape(1, 3 * H)
    bhh = b_hh.reshape(1, 3 * H)

    h = jnp.concatenate([x, jnp.zeros((NP - N, H), jnp.float32)])
    m = _mm(h, weight[0])
    for i in range(L):
        parts = _edge_agg(m, src2d, dst2d)
        if i < L - 1:
            h, m = _gru_mm(parts, h, wih, whh, bih, bhh, weight[i + 1])
        else:
            out = _gru_ro(parts, h, wih, whh, bih, bhh, batch3d)
    return out


# speedup vs baseline: 1.2968x; 1.2968x over previous
import functools

import jax
import jax.numpy as jnp
from jax import lax
from jax.experimental import pallas as pl
from jax.experimental.pallas import tpu as pltpu
from jax.experimental.pallas import tpu_sc as plsc

N = 10000
E = 320000
H = 128
G = 64
L = 3

NC = 2
NS = 16
NW = NC * NS
CHUNK = 128
CH = 80
E_PAD = NW * CH * CHUNK
NP = 10240
HN = NP // 2
ACC_R = HN + 128
LCAP = (CH + 1) * CHUNK + 16
CAPC = 2608
SENT = 1 << 20
DUM = HN
BLK = 1024
GRID = NP // BLK

_sc_mesh = plsc.VectorSubcoreMesh(core_axis_name="c", subcore_axis_name="s")


C2 = 64
NCH2 = E_PAD // C2
CPT2 = NCH2 // NW
QRT = 40
NBUF = 4


@functools.partial(
    pl.kernel,
    out_type=jax.ShapeDtypeStruct((NC, NP, H), jnp.float32),
    mesh=_sc_mesh,
    scratch_types=[
        pltpu.VMEM((QRT, C2), jnp.int32),
        pltpu.VMEM((QRT, C2), jnp.int32),
        pltpu.VMEM((NBUF, C2, H), jnp.float32),
        pltpu.VMEM_SHARED((NP, H), jnp.float32),
        pltpu.SemaphoreType.DMA,
        pltpu.SemaphoreType.DMA,
        pltpu.SemaphoreType.DMA,
        pltpu.SemaphoreType.DMA,
    ],
)
def _edge_agg(m_hbm, src_hbm, dst_hbm, out_hbm,
              sg_v, dg_v, rows_v, acc_sh, sem0, sem1, sem2, sem3):
    k = lax.axis_index("c")
    s = lax.axis_index("s")
    wid = k * NS + s
    sems = (sem0, sem1, sem2, sem3)

    zeros16 = jnp.zeros((16,), jnp.float32)

    def _zero_row(i, carry):
        for j in range(H // 16):
            rows_v[0, i, pl.ds(j * 16, 16)] = zeros16
        return carry

    lax.fori_loop(0, C2, _zero_row, 0)
    for q in range(NP // NS // C2):
        pltpu.sync_copy(rows_v.at[0],
                        acc_sh.at[pl.ds(s * (NP // NS) + q * C2, C2)])
    plsc.subcore_barrier()

    for quarter in range(CPT2 // QRT):
        base = wid * CPT2 + quarter * QRT
        pltpu.sync_copy(src_hbm.at[pl.ds(base, QRT)], sg_v)
        pltpu.sync_copy(dst_hbm.at[pl.ds(base, QRT)], dg_v)
        for b in range(NBUF):
            pltpu.async_copy(m_hbm.at[sg_v.at[b]], rows_v.at[b], sems[b])

        def _ring(g, carry):
            for b in range(NBUF):
                i = NBUF * g + b
                pltpu.make_async_copy(m_hbm.at[sg_v.at[i]], rows_v.at[b],
                                      sems[b]).wait()
                pltpu.sync_copy(rows_v.at[b], acc_sh.at[dg_v.at[i]],
                                add=True)

                @pl.when(i + NBUF < QRT)
                def _():
                    pltpu.async_copy(m_hbm.at[sg_v.at[i + NBUF]],
                                     rows_v.at[b], sems[b])

            return carry

        lax.fori_loop(0, QRT // NBUF, _ring, 0)
    plsc.subcore_barrier()

    nrows = NP // NS
    pltpu.sync_copy(acc_sh.at[pl.ds(s * nrows, nrows)],
                    out_hbm.at[k, pl.ds(s * nrows, nrows)])


def _mm_body(x_ref, w_ref, o_ref):
    o_ref[...] = jnp.dot(x_ref[...], w_ref[...],
                         preferred_element_type=jnp.float32)


_mm = pl.pallas_call(
    _mm_body,
    grid=(GRID,),
    in_specs=[
        pl.BlockSpec((BLK, H), lambda i: (i, 0)),
        pl.BlockSpec((H, H), lambda i: (0, 0)),
    ],
    out_specs=pl.BlockSpec((BLK, H), lambda i: (i, 0)),
    out_shape=jax.ShapeDtypeStruct((NP, H), jnp.float32),
)


def _gru(agg, h, wih, whh, bih, bhh):
    gi = jnp.dot(agg, wih, preferred_element_type=jnp.float32) + bih
    gh = jnp.dot(h, whh, preferred_element_type=jnp.float32) + bhh
    r = jax.nn.sigmoid(gi[:, :H] + gh[:, :H])
    z = jax.nn.sigmoid(gi[:, H:2 * H] + gh[:, H:2 * H])
    n = jnp.tanh(gi[:, 2 * H:] + r * gh[:, 2 * H:])
    return (1.0 - z) * n + z * h


_p0_spec = pl.BlockSpec((1, BLK, H), lambda i: (0, i, 0))
_p1_spec = pl.BlockSpec((1, BLK, H), lambda i: (1, i, 0))


def _gru_mm_body(p0_ref, p1_ref, h_ref, wih_ref, whh_ref, bih_ref, bhh_ref,
                 wn_ref, hn_ref, mn_ref):
    hn = _gru(p0_ref[0] + p1_ref[0], h_ref[...], wih_ref[...], whh_ref[...],
              bih_ref[...], bhh_ref[...])
    hn_ref[...] = hn
    mn_ref[...] = jnp.dot(hn, wn_ref[...], preferred_element_type=jnp.float32)


_gru_mm = pl.pallas_call(
    _gru_mm_body,
    grid=(GRID,),
    in_specs=[
        _p0_spec,
        _p1_spec,
        pl.BlockSpec((BLK, H), lambda i: (i, 0)),
        pl.BlockSpec((H, 3 * H), lambda i: (0, 0)),
        pl.BlockSpec((H, 3 * H), lambda i: (0, 0)),
        pl.BlockSpec((1, 3 * H), lambda i: (0, 0)),
        pl.BlockSpec((1, 3 * H), lambda i: (0, 0)),
        pl.BlockSpec((H, H), lambda i: (0, 0)),
    ],
    out_specs=[
        pl.BlockSpec((BLK, H), lambda i: (i, 0)),
        pl.BlockSpec((BLK, H), lambda i: (i, 0)),
    ],
    out_shape=[
        jax.ShapeDtypeStruct((NP, H), jnp.float32),
        jax.ShapeDtypeStruct((NP, H), jnp.float32),
    ],
)


def _gru_ro_body(p0_ref, p1_ref, h_ref, wih_ref, whh_ref, bih_ref, bhh_ref,
                 b_ref, out_ref):
    hn = _gru(p0_ref[0] + p1_ref[0], h_ref[...], wih_ref[...], whh_ref[...],
              bih_ref[...], bhh_ref[...])
    bid = b_ref[0, 0, :]
    oh = (bid[:, None] == lax.broadcasted_iota(jnp.int32, (BLK, G), 1)
          ).astype(jnp.float32)
    contrib = lax.dot_general(oh, hn, (((0,), (0,)), ((), ())),
                              preferred_element_type=jnp.float32)

    @pl.when(pl.program_id(0) == 0)
    def _():
        out_ref[...] = contrib

    @pl.when(pl.program_id(0) > 0)
    def _():
        out_ref[...] += contrib


_gru_ro = pl.pallas_call(
    _gru_ro_body,
    grid=(GRID,),
    in_specs=[
        _p0_spec,
        _p1_spec,
        pl.BlockSpec((BLK, H), lambda i: (i, 0)),
        pl.BlockSpec((H, 3 * H), lambda i: (0, 0)),
        pl.BlockSpec((H, 3 * H), lambda i: (0, 0)),
        pl.BlockSpec((1, 3 * H), lambda i: (0, 0)),
        pl.BlockSpec((1, 3 * H), lambda i: (0, 0)),
        pl.BlockSpec((1, 1, BLK), lambda i: (i, 0, 0)),
    ],
    out_specs=pl.BlockSpec((G, H), lambda i: (0, 0)),
    out_shape=jax.ShapeDtypeStruct((G, H), jnp.float32),
)


def kernel(x, edge_index, batch, weight, W_ih, W_hh, b_ih, b_hh):
    src = edge_index[0].astype(jnp.int32)
    dst = edge_index[1].astype(jnp.int32)
    pad = E_PAD - E
    src2d = jnp.concatenate([src, jnp.zeros((pad,), jnp.int32)]
                            ).reshape(NCH2, C2)
    dst2d = jnp.concatenate([dst, jnp.full((pad,), N, jnp.int32)]
                            ).reshape(NCH2, C2)
    batch3d = jnp.concatenate([batch.astype(jnp.int32),
                               jnp.full((NP - N,), G, jnp.int32)]
                              ).reshape(GRID, 1, BLK)

    wih = W_ih.T
    whh = W_hh.T
    bih = b_ih.reshape(1, 3 * H)
    bhh = b_hh.reshape(1, 3 * H)

    h = jnp.concatenate([x, jnp.zeros((NP - N, H), jnp.float32)])
    m = _mm(h, weight[0])
    for i in range(L):
        parts = _edge_agg(m, src2d, dst2d)
        if i < L - 1:
            h, m = _gru_mm(parts, parts, h, wih, whh, bih, bhh,
                           weight[i + 1])
        else:
            out = _gru_ro(parts, parts, h, wih, whh, bih, bhh, batch3d)
    return out
